# baseline (speedup 1.0000x reference)
import functools

import jax
import jax.numpy as jnp
from jax import lax
from jax.experimental import pallas as pl
from jax.experimental.pallas import tpu as pltpu, tpu_sc as plsc

NUM_CORES = 2
NUM_SUBCORES = 16
NUM_WORKERS = NUM_CORES * NUM_SUBCORES
BLK = 128
LANES = 16


@functools.partial(jax.jit, static_argnames=("n_rows", "n_tok", "dim"))
def _sc_gather(table, ids_t, *, n_rows, n_tok, dim):
    assert n_rows == NUM_WORKERS * BLK and dim == 32
    mesh = plsc.VectorSubcoreMesh(core_axis_name="c", subcore_axis_name="s")

    @functools.partial(
        pl.kernel,
        out_type=jax.ShapeDtypeStruct((n_tok, 4, NUM_WORKERS, 8, BLK), jnp.float32),
        mesh=mesh,
        scratch_types=[
            pltpu.VMEM((n_tok, BLK), jnp.int32),
            pltpu.VMEM((2, BLK, dim), jnp.float32),
            pltpu.VMEM((2, 4, 8, BLK), jnp.float32),
            pltpu.SemaphoreType.DMA,
            pltpu.SemaphoreType.DMA,
        ],
        compiler_params=pltpu.CompilerParams(
            use_tc_tiling_on_sc=False, needs_layout_passes=False
        ),
    )
    def k(table_hbm, idx_hbm, out_hbm, idx_v, rows_v, tbuf_v, sem_g, sem_o):
        wid = lax.axis_index("s") * NUM_CORES + lax.axis_index("c")
        pltpu.sync_copy(idx_hbm.at[:, pl.ds(wid * BLK, BLK)], idx_v)

        iota = lax.iota(jnp.int32, LANES)
        iotas = [iota + LANES * v for v in range(BLK // LANES)]

        def gather_start(j, b):
            return pltpu.async_copy(
                table_hbm.at[idx_v.at[j]], rows_v.at[b], sem_g
            )

        def gather_wait(b):
            pltpu.make_async_copy(
                table_hbm.at[idx_v.at[0]], rows_v.at[b], sem_g
            ).wait()

        def transpose(b):
            for k_ in range(BLK * dim // LANES):
                c = k_ >> 3
                col = jnp.full((LANES,), c, jnp.int32)
                val = plsc.load_gather(rows_v.at[b], [iotas[k_ & 7], col])
                tbuf_v[b, c >> 3, c & 7, pl.ds(LANES * (k_ & 7), LANES)] = val

        def out_start(j, b):
            for c8 in range(4):
                pltpu.async_copy(
                    tbuf_v.at[b, c8], out_hbm.at[j, c8, wid], sem_o
                )

        def out_wait(b):
            for c8 in range(4):
                pltpu.make_async_copy(
                    tbuf_v.at[b, c8], out_hbm.at[0, c8, wid], sem_o
                ).wait()

        gather_start(0, 0)

        def body(j, carry):
            b = j & 1
            gather_wait(b)
            gather_start(j + 1, 1 - b)
            out_wait(b)
            transpose(b)
            out_start(j, b)
            return carry

        gather_wait(0)
        gather_start(1, 1)
        transpose(0)
        out_start(0, 0)
        gather_wait(1)
        gather_start(2, 0)
        transpose(1)
        out_start(1, 1)
        lax.fori_loop(2, n_tok - 1, body, 0)
        b = (n_tok - 1) & 1
        gather_wait(b)
        out_wait(b)
        transpose(b)
        out_start(n_tok - 1, b)
        out_wait(1 - b)
        out_wait(b)

    return k(table, ids_t)


def kernel(token_ids, embedding_matrix):
    n_rows, n_tok = token_ids.shape
    dim = embedding_matrix.shape[1]
    ids_t = token_ids.astype(jnp.int32).T
    out5 = _sc_gather(embedding_matrix, ids_t, n_rows=n_rows, n_tok=n_tok, dim=dim)
    return out5.transpose(2, 4, 0, 1, 3).reshape(n_rows, n_tok, dim)


# speedup vs baseline: 1.0590x; 1.0590x over previous
import functools

import jax
import jax.numpy as jnp
from jax import lax
from jax.experimental import pallas as pl
from jax.experimental.pallas import tpu as pltpu, tpu_sc as plsc

NUM_CORES = 2
NUM_SUBCORES = 16
NUM_WORKERS = NUM_CORES * NUM_SUBCORES
BLK = 128
LANES = 16


@functools.partial(jax.jit, static_argnames=("n_rows", "n_tok", "dim"))
def _sc_gather(table, ids_t, *, n_rows, n_tok, dim):
    assert n_rows == NUM_WORKERS * BLK and dim == 32
    mesh = plsc.VectorSubcoreMesh(core_axis_name="c", subcore_axis_name="s")

    @functools.partial(
        pl.kernel,
        out_type=jax.ShapeDtypeStruct((n_tok, 4, NUM_WORKERS, 8, BLK), jnp.float32),
        mesh=mesh,
        scratch_types=[
            pltpu.VMEM((n_tok, BLK), jnp.int32),
            pltpu.VMEM((2, BLK, dim), jnp.float32),
            pltpu.VMEM((2, 4, 8, BLK), jnp.float32),
            pltpu.SemaphoreType.DMA,
            pltpu.SemaphoreType.DMA,
        ],
        compiler_params=pltpu.CompilerParams(
            use_tc_tiling_on_sc=False, needs_layout_passes=False
        ),
    )
    def k(table_hbm, idx_hbm, out_hbm, idx_v, rows_v, tbuf_v, sem_g, sem_o):
        wid = lax.axis_index("s") * NUM_CORES + lax.axis_index("c")
        pltpu.sync_copy(idx_hbm.at[:, pl.ds(wid * BLK, BLK)], idx_v)

        iota = lax.iota(jnp.int32, LANES)
        iotas = [iota + LANES * v for v in range(BLK // LANES)]
        cols = [jnp.full((LANES,), c, jnp.int32) for c in range(dim)]

        def gather_start(j, b):
            return pltpu.async_copy(
                table_hbm.at[idx_v.at[j]], rows_v.at[b], sem_g
            )

        def gather_wait(b):
            pltpu.make_async_copy(
                table_hbm.at[idx_v.at[0]], rows_v.at[b], sem_g
            ).wait()

        def transpose(b):
            for k_ in range(BLK * dim // LANES):
                c = k_ >> 3
                val = plsc.load_gather(rows_v.at[b], [iotas[k_ & 7], cols[c]])
                tbuf_v[b, c >> 3, c & 7, pl.ds(LANES * (k_ & 7), LANES)] = val

        def out_start(j, b):
            for c8 in range(4):
                pltpu.async_copy(
                    tbuf_v.at[b, c8], out_hbm.at[j, c8, wid], sem_o
                )

        def out_wait(b):
            for c8 in range(4):
                pltpu.make_async_copy(
                    tbuf_v.at[b, c8], out_hbm.at[0, c8, wid], sem_o
                ).wait()

        def stage(j, b, with_out_wait):
            gather_wait(b)
            if with_out_wait:
                out_wait(b)
            transpose(b)
            out_start(j, b)
            gather_start(j + 2, b)

        gather_start(0, 0)
        gather_start(1, 1)
        stage(0, 0, False)
        stage(1, 1, False)

        def body(t, carry):
            stage(2 * t, 0, True)
            stage(2 * t + 1, 1, True)
            return carry

        lax.fori_loop(1, n_tok // 2 - 1, body, 0)
        for j in (n_tok - 2, n_tok - 1):
            b = j & 1
            gather_wait(b)
            out_wait(b)
            transpose(b)
            out_start(j, b)
        out_wait(0)
        out_wait(1)

    return k(table, ids_t)


def kernel(token_ids, embedding_matrix):
    n_rows, n_tok = token_ids.shape
    dim = embedding_matrix.shape[1]
    ids_t = token_ids.astype(jnp.int32).T
    out5 = _sc_gather(embedding_matrix, ids_t, n_rows=n_rows, n_tok=n_tok, dim=dim)
    return out5.transpose(2, 4, 0, 1, 3).reshape(n_rows, n_tok, dim)


# speedup vs baseline: 1.3181x; 1.2446x over previous
import functools

import jax
import jax.numpy as jnp
from jax import lax
from jax.experimental import pallas as pl
from jax.experimental.pallas import tpu as pltpu, tpu_sc as plsc

NUM_CORES = 2
NUM_SUBCORES = 16
NUM_WORKERS = NUM_CORES * NUM_SUBCORES
BLK = 128
LANES = 16


@functools.partial(jax.jit, static_argnames=("n_rows", "n_tok", "dim"))
def _sc_gather(table, ids_t, *, n_rows, n_tok, dim):
    assert n_rows == NUM_WORKERS * BLK and dim == 32
    mesh = plsc.VectorSubcoreMesh(core_axis_name="c", subcore_axis_name="s")

    @functools.partial(
        pl.kernel,
        out_type=jax.ShapeDtypeStruct((n_tok, 4, NUM_WORKERS, 8 * BLK), jnp.float32),
        mesh=mesh,
        scratch_types=[
            pltpu.VMEM((n_tok, BLK), jnp.int32),
            pltpu.VMEM((2, BLK, dim), jnp.float32),
            pltpu.VMEM((2, 4 * 8 * BLK), jnp.float32),
            pltpu.SemaphoreType.DMA,
            pltpu.SemaphoreType.DMA,
        ],
        compiler_params=pltpu.CompilerParams(
            use_tc_tiling_on_sc=False, needs_layout_passes=False
        ),
    )
    def k(table_hbm, idx_hbm, out_hbm, idx_v, rows_v, tbuf_v, sem_g, sem_o):
        wid = lax.axis_index("s") * NUM_CORES + lax.axis_index("c")
        pltpu.sync_copy(idx_hbm.at[:, pl.ds(wid * BLK, BLK)], idx_v)

        iota = lax.iota(jnp.int32, LANES)

        def gather_start(j, b):
            return pltpu.async_copy(
                table_hbm.at[idx_v.at[j]], rows_v.at[b], sem_g
            )

        def gather_wait(b):
            pltpu.make_async_copy(
                table_hbm.at[idx_v.at[0]], rows_v.at[b], sem_g
            ).wait()

        def transpose(b):
            @plsc.parallel_loop(0, BLK * dim // LANES, 1, unroll=8)
            def _(k_):
                row = iota + ((k_ & 7) << 4)
                col = jnp.broadcast_to(k_ >> 3, (LANES,))
                val = plsc.load_gather(rows_v.at[b], [row, col])
                tbuf_v[b, pl.ds(k_ * LANES, LANES)] = val

        def out_start(j, b):
            for c8 in range(4):
                pltpu.async_copy(
                    tbuf_v.at[b, pl.ds(c8 * 8 * BLK, 8 * BLK)],
                    out_hbm.at[j, c8, wid],
                    sem_o,
                )

        def out_wait(b):
            for c8 in range(4):
                pltpu.make_async_copy(
                    tbuf_v.at[b, pl.ds(c8 * 8 * BLK, 8 * BLK)],
                    out_hbm.at[0, c8, wid],
                    sem_o,
                ).wait()

        def stage(j, b, with_out_wait):
            gather_wait(b)
            if with_out_wait:
                out_wait(b)
            transpose(b)
            out_start(j, b)
            gather_start(j + 2, b)

        gather_start(0, 0)
        gather_start(1, 1)
        stage(0, 0, False)
        stage(1, 1, False)

        def body(t, carry):
            stage(2 * t, 0, True)
            stage(2 * t + 1, 1, True)
            return carry

        lax.fori_loop(1, n_tok // 2 - 1, body, 0)
        for j in (n_tok - 2, n_tok - 1):
            b = j & 1
            gather_wait(b)
            out_wait(b)
            transpose(b)
            out_start(j, b)
        out_wait(0)
        out_wait(1)

    return k(table, ids_t)


def kernel(token_ids, embedding_matrix):
    n_rows, n_tok = token_ids.shape
    dim = embedding_matrix.shape[1]
    ids_t = token_ids.astype(jnp.int32).T
    out4 = _sc_gather(embedding_matrix, ids_t, n_rows=n_rows, n_tok=n_tok, dim=dim)
    out5 = out4.reshape(n_tok, 4, NUM_WORKERS, 8, BLK)
    return out5.transpose(2, 4, 0, 1, 3).reshape(n_rows, n_tok, dim)


# speedup vs baseline: 1.3420x; 1.0181x over previous
import functools

import jax
import jax.numpy as jnp
from jax import lax
from jax.experimental import pallas as pl
from jax.experimental.pallas import tpu as pltpu, tpu_sc as plsc

NUM_CORES = 2
NUM_SUBCORES = 16
NUM_WORKERS = NUM_CORES * NUM_SUBCORES
BLK = 128
LANES = 16


@functools.partial(jax.jit, static_argnames=("n_rows", "n_tok", "dim"))
def _sc_gather(table, ids_t, *, n_rows, n_tok, dim):
    assert n_rows == NUM_WORKERS * BLK and dim == 32
    mesh = plsc.VectorSubcoreMesh(core_axis_name="c", subcore_axis_name="s")

    @functools.partial(
        pl.kernel,
        out_type=jax.ShapeDtypeStruct((n_tok, 4, NUM_WORKERS, 8 * BLK), jnp.float32),
        mesh=mesh,
        scratch_types=[
            pltpu.VMEM((n_tok, BLK), jnp.int32),
            pltpu.VMEM((2, BLK, dim), jnp.float32),
            pltpu.VMEM((2, 4 * 8 * BLK), jnp.float32),
            pltpu.SemaphoreType.DMA,
            pltpu.SemaphoreType.DMA,
        ],
        compiler_params=pltpu.CompilerParams(
            use_tc_tiling_on_sc=False, needs_layout_passes=False
        ),
    )
    def k(table_hbm, idx_hbm, out_hbm, idx_v, rows_v, tbuf_v, sem_g, sem_o):
        wid = lax.axis_index("s") * NUM_CORES + lax.axis_index("c")
        pltpu.sync_copy(idx_hbm.at[:, pl.ds(wid * BLK, BLK)], idx_v)

        iota = lax.iota(jnp.int32, LANES)

        def gather_start(j, b):
            return pltpu.async_copy(
                table_hbm.at[idx_v.at[j]], rows_v.at[b], sem_g
            )

        def gather_wait(b):
            pltpu.make_async_copy(
                table_hbm.at[idx_v.at[0]], rows_v.at[b], sem_g
            ).wait()

        def transpose(b):
            @plsc.parallel_loop(0, BLK * dim // LANES, 1, unroll=32)
            def _(k_):
                row = iota + ((k_ & 7) << 4)
                col = jnp.broadcast_to(k_ >> 3, (LANES,))
                val = plsc.load_gather(rows_v.at[b], [row, col])
                tbuf_v[b, pl.ds(k_ * LANES, LANES)] = val

        def out_start(j, b):
            for c8 in range(4):
                pltpu.async_copy(
                    tbuf_v.at[b, pl.ds(c8 * 8 * BLK, 8 * BLK)],
                    out_hbm.at[j, c8, wid],
                    sem_o,
                )

        def out_wait(b):
            for c8 in range(4):
                pltpu.make_async_copy(
                    tbuf_v.at[b, pl.ds(c8 * 8 * BLK, 8 * BLK)],
                    out_hbm.at[0, c8, wid],
                    sem_o,
                ).wait()

        def stage(j, b, with_out_wait):
            gather_wait(b)
            if with_out_wait:
                out_wait(b)
            transpose(b)
            out_start(j, b)
            gather_start(j + 2, b)

        gather_start(0, 0)
        gather_start(1, 1)
        stage(0, 0, False)
        stage(1, 1, False)

        def body(t, carry):
            stage(2 * t, 0, True)
            stage(2 * t + 1, 1, True)
            return carry

        lax.fori_loop(1, n_tok // 2 - 1, body, 0)
        for j in (n_tok - 2, n_tok - 1):
            b = j & 1
            gather_wait(b)
            out_wait(b)
            transpose(b)
            out_start(j, b)
        out_wait(0)
        out_wait(1)

    return k(table, ids_t)


def kernel(token_ids, embedding_matrix):
    n_rows, n_tok = token_ids.shape
    dim = embedding_matrix.shape[1]
    ids_t = token_ids.astype(jnp.int32).T
    out4 = _sc_gather(embedding_matrix, ids_t, n_rows=n_rows, n_tok=n_tok, dim=dim)
    out5 = out4.reshape(n_tok, 4, NUM_WORKERS, 8, BLK)
    return out5.transpose(2, 4, 0, 1, 3).reshape(n_rows, n_tok, dim)
